# baseline (speedup 1.0000x reference)
import functools

import jax
import jax.numpy as jnp
from jax import lax
from jax.experimental import pallas as pl
from jax.experimental.pallas import tpu as pltpu
from jax.experimental.pallas import tpu_sc as plsc

LATENT = 128
_CHUNK = 128
_NC, _NS = 2, 16
_NW = _NC * _NS
_N_ACC = 10240


def _bf(x):
    return x.astype(jnp.bfloat16).astype(jnp.float32)


def _ln(h, gamma, beta):
    mu = jnp.mean(h, axis=-1, keepdims=True)
    var = jnp.mean((h - mu) ** 2, axis=-1, keepdims=True)
    return (h - mu) / jnp.sqrt(var + 1e-5) * gamma + beta


def _enc_body(x_ref, w1_ref, b1_ref, w2_ref, b2_ref, g_ref, bt_ref, o_ref):
    h = jnp.maximum(
        jnp.dot(x_ref[...], w1_ref[...], preferred_element_type=jnp.float32)
        + b1_ref[...], 0.0)
    h = jnp.dot(h, w2_ref[...], preferred_element_type=jnp.float32) + b2_ref[...]
    o_ref[...] = _ln(h, g_ref[...], bt_ref[...])


def _edge_body_mesh(e_ref, gs_ref, gr_ref, w1_ref, b1_ref, w2_ref, b2_ref,
                    gam_ref, bt_ref, o_ref):
    e = e_ref[...]
    cat = jnp.concatenate([e, gs_ref[...], gr_ref[...]], axis=-1)
    h = jnp.maximum(
        jnp.dot(cat, w1_ref[...], preferred_element_type=jnp.float32)
        + b1_ref[...], 0.0)
    h = jnp.dot(h, w2_ref[...], preferred_element_type=jnp.float32) + b2_ref[...]
    o_ref[...] = e + _ln(h, gam_ref[...], bt_ref[...])


def _edge_body_world(e_ref, gs_ref, gr_ref, w1_ref, b1_ref, w2_ref, b2_ref,
                     gam_ref, bt_ref, o_ref):
    e = e_ref[...]
    cat = jnp.concatenate([e, gs_ref[...], gr_ref[...]], axis=-1)
    h = jnp.maximum(
        jnp.dot(cat, w1_ref[...], preferred_element_type=jnp.float32) + b1_ref[...], 0.0)
    h = jnp.dot(h, w2_ref[...], preferred_element_type=jnp.float32) + b2_ref[...]
    o_ref[...] = e + _ln(h, gam_ref[...], bt_ref[...])


def _node_body(x_ref, am0_ref, am1_ref, aw0_ref, aw1_ref, w1_ref, b1_ref,
               w2_ref, b2_ref, gam_ref, bt_ref, o_ref):
    x = x_ref[...]
    am = am0_ref[...] + am1_ref[...]
    aw = aw0_ref[...] + aw1_ref[...]
    cat = jnp.concatenate([x, am, aw], axis=-1)
    h = jnp.maximum(
        jnp.dot(cat, w1_ref[...], preferred_element_type=jnp.float32)
        + b1_ref[...], 0.0)
    h = jnp.dot(h, w2_ref[...],
                preferred_element_type=jnp.float32,
                ) + b2_ref[...]
    o_ref[...] = x + _ln(h, gam_ref[...], bt_ref[...])


def _proj_body(x_ref, w_ref, os_ref, or_ref):
    z = jnp.dot(x_ref[...], w_ref[...], preferred_element_type=jnp.float32)
    os_ref[0] = z[:, :LATENT]
    or_ref[0] = z[:, LATENT:2 * LATENT]
    os_ref[1] = z[:, 2 * LATENT:3 * LATENT]
    or_ref[1] = z[:, 3 * LATENT:]


def _dec_body(x_ref, w1_ref, b1_ref, w2_ref, b2_ref, o_ref):
    h = jnp.maximum(
        jnp.dot(x_ref[...], w1_ref[...],
                preferred_element_type=jnp.float32,
                ) + b1_ref[...], 0.0)
    o_ref[...] = (jnp.dot(h, w2_ref[...],
                          preferred_element_type=jnp.float32,
                          ) + b2_ref[...])


def _param_spec(shape):
    return pl.BlockSpec(shape, lambda i: (0,) * len(shape))


def _rows_spec(blk, ncols):
    return pl.BlockSpec((blk, ncols), lambda i: (i, 0))


def _call_rows(body, n_rows, blk, row_args, row_widths, param_args, out_cols,
               interpret=False, row_offsets=None):
    grid = n_rows // blk
    offs = row_offsets or [0] * len(row_args)
    in_specs = ([pl.BlockSpec((blk, w),
                              functools.partial(lambda o, i: (i + o, 0), o))
                 for w, o in zip(row_widths, offs)]
                + [_param_spec(p.shape) for p in param_args])
    return pl.pallas_call(
        body,
        grid=(grid,),
        in_specs=in_specs,
        out_specs=_rows_spec(blk, out_cols),
        out_shape=jax.ShapeDtypeStruct((n_rows, out_cols), jnp.float32),
        interpret=interpret,
    )(*row_args, *param_args)


def _call_proj(node, wstack, n, blk, interpret=False):
    grid = n // blk
    spec = pl.BlockSpec((2, blk, LATENT), lambda i: (0, i, 0))
    return pl.pallas_call(
        _proj_body,
        grid=(grid,),
        in_specs=[_rows_spec(blk, LATENT), _param_spec(wstack.shape)],
        out_specs=[spec, spec],
        out_shape=[jax.ShapeDtypeStruct((2, n, LATENT), jnp.float32)] * 2,
        interpret=interpret,
    )(node, wstack)


def _sc_mesh():
    return plsc.VectorSubcoreMesh(core_axis_name="c", subcore_axis_name="s",
                                  num_cores=_NC, num_subcores=_NS)


def _idx_spec():
    return pl.BlockSpec((1, _CHUNK), lambda i: (0, i))


def _row_blk_spec():
    return pl.BlockSpec((_CHUNK, LATENT), lambda i: (i, 0))


def _sc_gather_sum(zs, zr, i_s, i_r, e_tot):
    f32 = jnp.float32

    @functools.partial(
        pl.kernel,
        out_type=[jax.ShapeDtypeStruct((e_tot, LATENT), f32),
                  jax.ShapeDtypeStruct((e_tot, LATENT), f32)],
        mesh=_sc_mesh(),
        scratch_types=[pltpu.SemaphoreType.DMA, pltpu.SemaphoreType.DMA],
    )
    def k(zs_h, zr_h, is_h, ir_h, gs_h, gr_h, sem1, sem2):
        def body(is_ref, ir_ref, os_ref, or_ref):
            c1 = pltpu.async_copy(zs_h.at[is_ref.at[0]], os_ref, sem1)
            c2 = pltpu.async_copy(zr_h.at[ir_ref.at[0]], or_ref, sem2)
            c1.wait()
            c2.wait()

        pltpu.emit_pipeline(
            body,
            grid=(e_tot // _CHUNK,),
            in_specs=[_idx_spec(), _idx_spec()],
            out_specs=[_row_blk_spec(), _row_blk_spec()],
            core_axis_name=("c", "s"),
            dimension_semantics=(pltpu.PARALLEL,),
        )(is_h, ir_h, gs_h, gr_h)

    return k(zs, zr, i_s, i_r)


def _sc_segment_sum(me2, we2, i_m, i_w, em_pad, ew_pad):
    f32 = jnp.float32
    per_tile = _N_ACC // _NS
    mesh_chunks = em_pad // _CHUNK

    @functools.partial(
        pl.kernel,
        out_type=[jax.ShapeDtypeStruct((_NC, _N_ACC, LATENT), f32),
                  jax.ShapeDtypeStruct((_NC, _N_ACC, LATENT), f32)],
        mesh=_sc_mesh(),
        scratch_types=[pltpu.VMEM_SHARED((_N_ACC, LATENT), f32)],
    )
    def k(pm_h, pw_h, im_h, iw_h, z_h, om_h, ow_h, agg):
        cidx = lax.axis_index("c")
        sidx = lax.axis_index("s")
        base = sidx * per_tile

        def zero_agg():
            pltpu.sync_copy(z_h.at[pl.ds(base, per_tile)],
                            agg.at[pl.ds(base, per_tile)])

        def scatter(p_h, i_h, n_chunks):
            def body(x_ref, i_ref):
                pltpu.sync_copy(x_ref, agg.at[i_ref.at[0]], add=True)

            pltpu.emit_pipeline(
                body,
                grid=(n_chunks,),
                in_specs=[_row_blk_spec(), _idx_spec()],
                out_specs=[],
                core_axis_name=("c", "s"),
                dimension_semantics=(pltpu.PARALLEL,),
            )(p_h, i_h)

        def dump(o_h):
            @pl.loop(0, per_tile // _CHUNK)
            def _(j):
                off = base + j * _CHUNK
                pltpu.sync_copy(agg.at[pl.ds(off, _CHUNK)],
                                o_h.at[cidx, pl.ds(off, _CHUNK)])

        zero_agg()
        plsc.subcore_barrier()
        scatter(pm_h, im_h, mesh_chunks)
        plsc.subcore_barrier()
        dump(om_h)
        plsc.subcore_barrier()
        zero_agg()
        plsc.subcore_barrier()
        scatter(pw_h, iw_h, ew_pad // _CHUNK)
        plsc.subcore_barrier()
        dump(ow_h)

    return k(me2, we2, i_m, i_w, jnp.zeros((_N_ACC, LATENT), f32))


def kernel(node_features, mesh_edge_features, world_edge_features,
           mesh_senders, mesh_receivers, world_senders, world_receivers,
           params, *, interpret=False):
    n = node_features.shape[0]
    em = mesh_edge_features.shape[0]
    ew = world_edge_features.shape[0]
    span = _CHUNK * _NW
    em_pad = ((em + span - 1) // span) * span
    ew_pad = ((ew + span - 1) // span) * span
    e_tot = em_pad + ew_pad

    f32 = jnp.float32

    def pad_idx(ix, tot, fill):
        ix = ix.astype(jnp.int32)
        return jnp.pad(ix, (0, tot - ix.shape[0]), constant_values=fill)

    i_s = jnp.concatenate([pad_idx(mesh_senders, em_pad, 0),
                           pad_idx(world_senders, ew_pad, 0)]
                          ).reshape(1, e_tot)
    i_r = jnp.concatenate([pad_idx(mesh_receivers, em_pad, 0),
                           pad_idx(world_receivers, ew_pad, 0)]
                          ).reshape(1, e_tot)
    imr_s = pad_idx(mesh_receivers, em_pad, n).reshape(1, em_pad)
    iwr_s = pad_idx(world_receivers, ew_pad, n).reshape(1, ew_pad)

    def pad_cols(x, to):
        return jnp.pad(x, ((0, 0), (0, to - x.shape[1])))

    def fold_norm(p, mean, std):
        w1 = p["W1"] / std[:, None]
        b1 = p["b1"] - (mean / std) @ p["W1"]
        return w1, b1

    BN = 2000

    def enc(x, w1, b1, p):
        din = x.shape[1]
        dpad = 16
        xp = pad_cols(x.astype(f32), dpad)
        w1p = jnp.pad(w1, ((0, dpad - din), (0, 0)))
        pa = [w1p, b1.reshape(1, -1), p["W2"], p["b2"].reshape(1, -1),
              p["g"].reshape(1, -1), p["beta"].reshape(1, -1)]
        return _call_rows(_enc_body, x.shape[0], BN, [xp], [dpad], pa, LATENT,
                          interpret=interpret)

    pe = params["enc_node"]
    node = enc(node_features, pe["W1"], pe["b1"], pe)
    pm_ = params["enc_mesh"]
    w1m, b1m = fold_norm(pm_, params["mesh_norm_mean"], params["mesh_norm_std"])
    me = enc(mesh_edge_features, w1m, b1m, pm_)
    pw_ = params["enc_world"]
    w1w, b1w = fold_norm(pw_, params["world_norm_mean"], params["world_norm_std"])
    we = enc(world_edge_features, w1w, b1w, pw_)

    me_lat = jnp.pad(me, ((0, em_pad - em), (0, 0)))
    we_lat = jnp.pad(we, ((0, ew_pad - ew), (0, 0)))

    steps = params["proc_mesh"]["W1"].shape[0]
    BE = 4096
    BE_W = 2048

    def split_w1(w1):
        return w1[:LATENT], w1[LATENT:2 * LATENT], w1[2 * LATENT:]

    for i in range(steps):
        pmesh = jax.tree_util.tree_map(lambda a: a[i], params["proc_mesh"])
        pworld = jax.tree_util.tree_map(lambda a: a[i], params["proc_world"])
        pnode = jax.tree_util.tree_map(lambda a: a[i], params["proc_node"])

        _DBG_XLA = False
        if _DBG_XLA:
            gs = node[i_s[0]]
            gr = node[i_r[0]]
        else:
            gs, gr = _sc_gather_sum(node, node, i_s, i_r, e_tot)

        def edge_params(p):
            return [p["W1"], p["b1"].reshape(1, -1), p["W2"],
                    p["b2"].reshape(1, -1), p["g"].reshape(1, -1),
                    p["beta"].reshape(1, -1)]

        me_lat = _call_rows(_edge_body_mesh, em_pad, BE,
                            [me_lat, gs, gr], [LATENT] * 3,
                            edge_params(pmesh), LATENT, interpret=interpret)
        woff = em_pad // BE_W
        we_lat = _call_rows(_edge_body_world, ew_pad, BE_W,
                            [we_lat, gs, gr], [LATENT] * 3,
                            edge_params(pworld), LATENT, interpret=interpret,
                            row_offsets=[0, woff, woff])

        if _DBG_XLA:
            am_x = jax.ops.segment_sum(me_lat, imr_s[0], num_segments=_N_ACC)
            aw_x = jax.ops.segment_sum(we_lat, iwr_s[0], num_segments=_N_ACC)
            zero = jnp.zeros((_N_ACC, LATENT), f32)
            am = jnp.stack([am_x, zero])
            aw = jnp.stack([aw_x, zero])
        else:
            am, aw = _sc_segment_sum(me_lat, we_lat, imr_s, iwr_s,
                                     em_pad, ew_pad)

        pa_n = [pnode["W1"], pnode["b1"].reshape(1, -1), pnode["W2"],
                pnode["b2"].reshape(1, -1), pnode["g"].reshape(1, -1),
                pnode["beta"].reshape(1, -1)]
        node = _call_rows(_node_body, n, BN,
                          [node, am[0], am[1], aw[0], aw[1]], [LATENT] * 5,
                          pa_n, LATENT, interpret=interpret)

    pd = params["dec"]
    out_dim = pd["W2"].shape[1]
    w2p = pad_cols(pd["W2"], LATENT)
    b2p = jnp.pad(pd["b2"], (0, LATENT - out_dim))
    pa_d = [pd["W1"], pd["b1"].reshape(1, -1), w2p, b2p.reshape(1, -1)]
    out = _call_rows(_dec_body, n, BN, [node], [LATENT], pa_d, LATENT,
                     interpret=interpret)
    return out[:, :out_dim]
